# baseline (speedup 1.0000x reference)
import functools

import jax
import jax.numpy as jnp
from jax import lax
from jax.experimental import pallas as pl
from jax.experimental.pallas import tpu as pltpu
from jax.experimental.pallas import tpu_sc as plsc

_NC = 2
_NS = 16
_NW = _NC * _NS
_LANES = 16
_WAVE = 8
_TILE = 128


def _sc_gather(uidx, iidx, utab_t, itab_t):
    batch = uidx.shape[0]
    emb_dim = utab_t.shape[0]
    b_per_w = batch // _NW
    n_waves = b_per_w // _WAVE

    mesh = plsc.VectorSubcoreMesh(core_axis_name="c", subcore_axis_name="s")

    @functools.partial(
        pl.kernel,
        mesh=mesh,
        out_type=[
            jax.ShapeDtypeStruct((emb_dim, batch), jnp.float32),
            jax.ShapeDtypeStruct((emb_dim, batch), jnp.float32),
        ],
        scratch_types=[
            pltpu.VMEM((b_per_w + _LANES,), jnp.int32),
            pltpu.VMEM((b_per_w + _LANES,), jnp.int32),
            pltpu.VMEM((4, _WAVE, emb_dim, _TILE), jnp.float32),
            pltpu.VMEM((emb_dim, b_per_w), jnp.float32),
            pltpu.VMEM((emb_dim, b_per_w), jnp.float32),
            pltpu.SemaphoreType.DMA,
            pltpu.SemaphoreType.DMA,
            pltpu.SemaphoreType.DMA,
            pltpu.SemaphoreType.DMA,
        ],
        compiler_params=pltpu.CompilerParams(use_tc_tiling_on_sc=True,
                                             needs_layout_passes=False),
    )
    def gather_kernel(uidx_hbm, iidx_hbm, utab_hbm, itab_hbm,
                      uout_hbm, iout_hbm,
                      uidx_v, iidx_v, blk_v, ucol_v, icol_v,
                      sem0, sem1, sem2, sem3):
        wid = lax.axis_index("s") * _NC + lax.axis_index("c")
        base = wid * b_per_w
        pltpu.sync_copy(uidx_hbm.at[pl.ds(base, b_per_w)],
                        uidx_v.at[pl.ds(0, b_per_w)])
        pltpu.sync_copy(iidx_hbm.at[pl.ds(base, b_per_w)],
                        iidx_v.at[pl.ds(0, b_per_w)])

        lane = lax.broadcasted_iota(jnp.int32, (_LANES,), 0)

        def one_table(tab_hbm, idx_v, col_v):
            def fire(w, slot, sem):
                vec = idx_v[pl.ds(w * _WAVE, _LANES)]
                for k in range(_WAVE):
                    r = jnp.max(jnp.where(lane == k, vec, 0))
                    r_al = pl.multiple_of((r >> 7) << 7, _TILE)
                    pltpu.async_copy(tab_hbm.at[:, pl.ds(r_al, _TILE)],
                                     blk_v.at[slot, k], sem)

            def drain(slot, sem):
                for k in range(_WAVE):
                    pltpu.make_async_copy(tab_hbm.at[:, pl.ds(0, _TILE)],
                                          blk_v.at[slot, k], sem).wait()

            def extract(w, slot):
                vec = idx_v[pl.ds(w * _WAVE, _LANES)]
                for k in range(_WAVE):
                    r = jnp.max(jnp.where(lane == k, vec, 0))
                    col_splat = (lane * 0) + (r & (_TILE - 1))
                    vals = plsc.load_gather(
                        blk_v, [lane * 0 + slot, lane * 0 + k, lane,
                                col_splat])
                    pos_splat = lane * 0 + (w * _WAVE + k)
                    plsc.store_scatter(col_v, [lane, pos_splat], vals)

            sems = (sem0, sem1, sem2, sem3)
            n_slots = 4
            ahead = n_slots - 1
            for p in range(ahead):
                fire(p, p, sems[p])

            def body(gS, carry):
                w = n_slots * gS
                for j in range(n_slots):
                    fs = (j + ahead) % n_slots
                    fire(w + j + ahead, fs, sems[fs])
                    drain(j, sems[j])
                    extract(w + j, j)
                return carry

            n_body = (n_waves - ahead) // n_slots
            lax.fori_loop(0, n_body, body, 0)
            w_tail = n_slots * n_body
            for j in range(n_waves - w_tail):
                fs = (j + ahead) % n_slots
                if w_tail + j + ahead < n_waves:
                    fire(w_tail + j + ahead, fs, sems[fs])
                drain(j % n_slots, sems[j % n_slots])
                extract(w_tail + j, j % n_slots)

        one_table(utab_hbm, uidx_v, ucol_v)
        one_table(itab_hbm, iidx_v, icol_v)
        out_off = pl.multiple_of(base, _TILE)
        pltpu.sync_copy(ucol_v, uout_hbm.at[:, pl.ds(out_off, b_per_w)])
        pltpu.sync_copy(icol_v, iout_hbm.at[:, pl.ds(out_off, b_per_w)])

    return gather_kernel(uidx, iidx, utab_t, itab_t)


def _mlp_kernel_body(ue_ref, ie_ref, w1u_ref, w1i_ref, b1_ref,
                     w2_ref, b2_ref, w3_ref, b3_ref, o_ref):
    dn = (((0,), (0,)), ((), ()))
    h = lax.dot_general(ue_ref[...], w1u_ref[...], dn,
                        preferred_element_type=jnp.float32)
    h += lax.dot_general(ie_ref[...], w1i_ref[...], dn,
                         preferred_element_type=jnp.float32)
    h = jnp.maximum(h + b1_ref[...], 0.0)
    h = jnp.dot(h, w2_ref[...], preferred_element_type=jnp.float32)
    h = jnp.maximum(h + b2_ref[...], 0.0)
    z = lax.dot_general(w3_ref[...], h, (((0,), (1,)), ((), ())),
                        preferred_element_type=jnp.float32)
    z = z + b3_ref[...]
    o_ref[...] = 1.0 / (1.0 + jnp.exp(-z))


def _tc_mlp(ue_t, ie_t, W1u, W1i, b1, W2, b2, W3, b3, block_b=2048):
    emb_dim, batch = ue_t.shape
    grid_n = batch // block_b
    full = lambda shape: pl.BlockSpec(shape, lambda i: (0,) * len(shape))
    return pl.pallas_call(
        _mlp_kernel_body,
        grid=(grid_n,),
        in_specs=[
            pl.BlockSpec((emb_dim, block_b), lambda i: (0, i)),
            pl.BlockSpec((emb_dim, block_b), lambda i: (0, i)),
            full(W1u.shape),
            full(W1i.shape),
            full(b1.shape),
            full(W2.shape),
            full(b2.shape),
            full(W3.shape),
            full(b3.shape),
        ],
        out_specs=pl.BlockSpec((1, block_b), lambda i: (0, i)),
        out_shape=jax.ShapeDtypeStruct((1, batch), jnp.float32),
    )(ue_t, ie_t, W1u, W1i, b1, W2, b2, W3, b3)


def kernel(user_input, item_input, user_table, item_table,
           W1, b1, W2, b2, W3, b3):
    emb_dim = user_table.shape[1]

    uidx = user_input.astype(jnp.int32)
    iidx = item_input.astype(jnp.int32)

    ue_t, ie_t = _sc_gather(uidx, iidx, user_table.T, item_table.T)

    W1u = W1[:emb_dim]
    W1i = W1[emb_dim:]
    out_t = _tc_mlp(ue_t, ie_t, W1u, W1i,
                    b1.reshape(1, -1), W2, b2.reshape(1, -1),
                    W3, b3.reshape(1, -1))
    return out_t.T


# speedup vs baseline: 1.0127x; 1.0127x over previous
import functools

import jax
import jax.numpy as jnp
from jax import lax
from jax.experimental import pallas as pl
from jax.experimental.pallas import tpu as pltpu
from jax.experimental.pallas import tpu_sc as plsc

_NC = 2
_NS = 16
_NW = _NC * _NS
_LANES = 16
_WAVE = 8
_TILE = 128


def _sc_gather(uidx, iidx, utab_t, itab_t):
    batch = uidx.shape[0]
    emb_dim = utab_t.shape[0]
    b_per_w = batch // _NW
    n_waves = b_per_w // _WAVE

    mesh = plsc.VectorSubcoreMesh(core_axis_name="c", subcore_axis_name="s")

    @functools.partial(
        pl.kernel,
        mesh=mesh,
        out_type=[
            jax.ShapeDtypeStruct((emb_dim, batch), jnp.float32),
            jax.ShapeDtypeStruct((emb_dim, batch), jnp.float32),
        ],
        scratch_types=[
            pltpu.VMEM((b_per_w + _LANES,), jnp.int32),
            pltpu.VMEM((b_per_w + _LANES,), jnp.int32),
            pltpu.VMEM((4, _WAVE, emb_dim, _TILE), jnp.float32),
            pltpu.VMEM((emb_dim, b_per_w), jnp.float32),
            pltpu.VMEM((emb_dim, b_per_w), jnp.float32),
            pltpu.SemaphoreType.DMA,
            pltpu.SemaphoreType.DMA,
            pltpu.SemaphoreType.DMA,
            pltpu.SemaphoreType.DMA,
        ],
        compiler_params=pltpu.CompilerParams(use_tc_tiling_on_sc=True,
                                             needs_layout_passes=False),
    )
    def gather_kernel(uidx_hbm, iidx_hbm, utab_hbm, itab_hbm,
                      uout_hbm, iout_hbm,
                      uidx_v, iidx_v, blk_v, ucol_v, icol_v,
                      sem0, sem1, sem2, sem3):
        wid = lax.axis_index("s") * _NC + lax.axis_index("c")
        base = wid * b_per_w
        pltpu.sync_copy(uidx_hbm.at[pl.ds(base, b_per_w)],
                        uidx_v.at[pl.ds(0, b_per_w)])
        pltpu.sync_copy(iidx_hbm.at[pl.ds(base, b_per_w)],
                        iidx_v.at[pl.ds(0, b_per_w)])

        lane = lax.broadcasted_iota(jnp.int32, (_LANES,), 0)

        def one_table(tab_hbm, idx_v, col_v):
            def fire(w, slot, sem):
                vec = idx_v[pl.ds(w * _WAVE, _LANES)]
                for k in range(_WAVE):
                    r = jnp.max(jnp.where(lane == k, vec, 0))
                    r_al = pl.multiple_of((r >> 7) << 7, _TILE)
                    pltpu.async_copy(tab_hbm.at[:, pl.ds(r_al, _TILE)],
                                     blk_v.at[slot, k], sem)

            def drain(slot, sem):
                for k in range(_WAVE):
                    pltpu.make_async_copy(tab_hbm.at[:, pl.ds(0, _TILE)],
                                          blk_v.at[slot, k], sem).wait()

            def extract(w, slot):
                vec = idx_v[pl.ds(w * _WAVE, _LANES)]
                for k in range(_WAVE):
                    r = jnp.max(jnp.where(lane == k, vec, 0))
                    col_splat = (lane * 0) + (r & (_TILE - 1))
                    vals = plsc.load_gather(
                        blk_v, [lane * 0 + slot, lane * 0 + k, lane,
                                col_splat])
                    pos_splat = lane * 0 + (w * _WAVE + k)
                    plsc.store_scatter(col_v, [lane, pos_splat], vals)

            sems = (sem0, sem1, sem2, sem3)
            n_slots = 4
            ahead = n_slots - 1
            for p in range(ahead):
                fire(p, p, sems[p])

            def body(gS, carry):
                w = n_slots * gS
                for j in range(n_slots):
                    fs = (j + ahead) % n_slots
                    fire(w + j + ahead, fs, sems[fs])
                    drain(j, sems[j])
                    extract(w + j, j)
                return carry

            n_body = (n_waves - ahead) // n_slots
            lax.fori_loop(0, n_body, body, 0)
            w_tail = n_slots * n_body
            for j in range(n_waves - w_tail):
                fs = (j + ahead) % n_slots
                if w_tail + j + ahead < n_waves:
                    fire(w_tail + j + ahead, fs, sems[fs])
                drain(j % n_slots, sems[j % n_slots])
                extract(w_tail + j, j % n_slots)

        one_table(utab_hbm, uidx_v, ucol_v)
        one_table(itab_hbm, iidx_v, icol_v)
        out_off = pl.multiple_of(base, _TILE)
        pltpu.sync_copy(ucol_v, uout_hbm.at[:, pl.ds(out_off, b_per_w)])
        pltpu.sync_copy(icol_v, iout_hbm.at[:, pl.ds(out_off, b_per_w)])

    return gather_kernel(uidx, iidx, utab_t, itab_t)


def _mlp_kernel_body(ue_ref, ie_ref, w1u_ref, w1i_ref, b1_ref,
                     w2_ref, b2_ref, w3_ref, b3_ref, o_ref):
    dn = (((0,), (0,)), ((), ()))
    h = lax.dot_general(ue_ref[...], w1u_ref[...], dn,
                        preferred_element_type=jnp.float32)
    h += lax.dot_general(ie_ref[...], w1i_ref[...], dn,
                         preferred_element_type=jnp.float32)
    h = jnp.maximum(h + b1_ref[...], 0.0)
    h = jnp.dot(h, w2_ref[...], preferred_element_type=jnp.float32)
    h = jnp.maximum(h + b2_ref[...], 0.0)
    z = lax.dot_general(w3_ref[...], h, (((0,), (1,)), ((), ())),
                        preferred_element_type=jnp.float32)
    z = z + b3_ref[...]
    o_ref[...] = 1.0 / (1.0 + jnp.exp(-z))


def _tc_mlp(ue_t, ie_t, W1u, W1i, b1, W2, b2, W3, b3, block_b=8192):
    emb_dim, batch = ue_t.shape
    grid_n = batch // block_b
    full = lambda shape: pl.BlockSpec(shape, lambda i: (0,) * len(shape))
    return pl.pallas_call(
        _mlp_kernel_body,
        grid=(grid_n,),
        in_specs=[
            pl.BlockSpec((emb_dim, block_b), lambda i: (0, i)),
            pl.BlockSpec((emb_dim, block_b), lambda i: (0, i)),
            full(W1u.shape),
            full(W1i.shape),
            full(b1.shape),
            full(W2.shape),
            full(b2.shape),
            full(W3.shape),
            full(b3.shape),
        ],
        out_specs=pl.BlockSpec((1, block_b), lambda i: (0, i)),
        out_shape=jax.ShapeDtypeStruct((1, batch), jnp.float32),
    )(ue_t, ie_t, W1u, W1i, b1, W2, b2, W3, b3)


def kernel(user_input, item_input, user_table, item_table,
           W1, b1, W2, b2, W3, b3):
    emb_dim = user_table.shape[1]

    uidx = user_input.astype(jnp.int32)
    iidx = item_input.astype(jnp.int32)

    ue_t, ie_t = _sc_gather(uidx, iidx, user_table.T, item_table.T)

    W1u = W1[:emb_dim]
    W1i = W1[emb_dim:]
    out_t = _tc_mlp(ue_t, ie_t, W1u, W1i,
                    b1.reshape(1, -1), W2, b2.reshape(1, -1),
                    W3, b3.reshape(1, -1))
    return out_t.T


# speedup vs baseline: 1.0318x; 1.0189x over previous
import functools

import jax
import jax.numpy as jnp
from jax import lax
from jax.experimental import pallas as pl
from jax.experimental.pallas import tpu as pltpu
from jax.experimental.pallas import tpu_sc as plsc

_NC = 2
_NS = 16
_NW = _NC * _NS
_LANES = 16
_WAVE = 8
_TILE = 128


def _sc_gather(uidx, iidx, utab_t, itab_t):
    batch = uidx.shape[0]
    emb_dim = utab_t.shape[0]
    b_per_w = batch // _NW
    n_waves = b_per_w // _WAVE

    mesh = plsc.VectorSubcoreMesh(core_axis_name="c", subcore_axis_name="s")

    @functools.partial(
        pl.kernel,
        mesh=mesh,
        out_type=[
            jax.ShapeDtypeStruct((emb_dim, batch), jnp.float32),
            jax.ShapeDtypeStruct((emb_dim, batch), jnp.float32),
        ],
        scratch_types=[
            pltpu.VMEM((b_per_w + _LANES,), jnp.int32),
            pltpu.VMEM((b_per_w + _LANES,), jnp.int32),
            pltpu.VMEM((6, _WAVE, emb_dim, _TILE), jnp.float32),
            pltpu.VMEM((emb_dim, b_per_w), jnp.float32),
            pltpu.VMEM((emb_dim, b_per_w), jnp.float32),
            pltpu.SemaphoreType.DMA,
            pltpu.SemaphoreType.DMA,
            pltpu.SemaphoreType.DMA,
            pltpu.SemaphoreType.DMA,
            pltpu.SemaphoreType.DMA,
            pltpu.SemaphoreType.DMA,
        ],
        compiler_params=pltpu.CompilerParams(use_tc_tiling_on_sc=True,
                                             needs_layout_passes=False),
    )
    def gather_kernel(uidx_hbm, iidx_hbm, utab_hbm, itab_hbm,
                      uout_hbm, iout_hbm,
                      uidx_v, iidx_v, blk_v, ucol_v, icol_v,
                      sem0, sem1, sem2, sem3, sem4, sem5):
        wid = lax.axis_index("s") * _NC + lax.axis_index("c")
        base = wid * b_per_w
        pltpu.sync_copy(uidx_hbm.at[pl.ds(base, b_per_w)],
                        uidx_v.at[pl.ds(0, b_per_w)])
        pltpu.sync_copy(iidx_hbm.at[pl.ds(base, b_per_w)],
                        iidx_v.at[pl.ds(0, b_per_w)])

        lane = lax.broadcasted_iota(jnp.int32, (_LANES,), 0)

        def one_table(tab_hbm, idx_v, col_v):
            def fire(w, slot, sem):
                vec = idx_v[pl.ds(w * _WAVE, _LANES)]
                for k in range(_WAVE):
                    r = jnp.max(jnp.where(lane == k, vec, 0))
                    r_al = pl.multiple_of((r >> 7) << 7, _TILE)
                    pltpu.async_copy(tab_hbm.at[:, pl.ds(r_al, _TILE)],
                                     blk_v.at[slot, k], sem)

            def drain(slot, sem):
                for k in range(_WAVE):
                    pltpu.make_async_copy(tab_hbm.at[:, pl.ds(0, _TILE)],
                                          blk_v.at[slot, k], sem).wait()

            def extract(w, slot):
                vec = idx_v[pl.ds(w * _WAVE, _LANES)]
                for k in range(_WAVE):
                    r = jnp.max(jnp.where(lane == k, vec, 0))
                    col_splat = (lane * 0) + (r & (_TILE - 1))
                    vals = plsc.load_gather(
                        blk_v, [lane * 0 + slot, lane * 0 + k, lane,
                                col_splat])
                    pos_splat = lane * 0 + (w * _WAVE + k)
                    plsc.store_scatter(col_v, [lane, pos_splat], vals)

            sems = (sem0, sem1, sem2, sem3, sem4, sem5)
            n_slots = 6
            ahead = n_slots - 1
            for p in range(ahead):
                fire(p, p, sems[p])

            def body(gS, carry):
                w = n_slots * gS
                for j in range(n_slots):
                    fs = (j + ahead) % n_slots
                    fire(w + j + ahead, fs, sems[fs])
                    drain(j, sems[j])
                    extract(w + j, j)
                return carry

            n_body = (n_waves - ahead) // n_slots
            lax.fori_loop(0, n_body, body, 0)
            w_tail = n_slots * n_body
            for j in range(n_waves - w_tail):
                fs = (j + ahead) % n_slots
                if w_tail + j + ahead < n_waves:
                    fire(w_tail + j + ahead, fs, sems[fs])
                drain(j % n_slots, sems[j % n_slots])
                extract(w_tail + j, j % n_slots)

        one_table(utab_hbm, uidx_v, ucol_v)
        one_table(itab_hbm, iidx_v, icol_v)
        out_off = pl.multiple_of(base, _TILE)
        pltpu.sync_copy(ucol_v, uout_hbm.at[:, pl.ds(out_off, b_per_w)])
        pltpu.sync_copy(icol_v, iout_hbm.at[:, pl.ds(out_off, b_per_w)])

    return gather_kernel(uidx, iidx, utab_t, itab_t)


def _mlp_kernel_body(ue_ref, ie_ref, w1u_ref, w1i_ref, b1_ref,
                     w2_ref, b2_ref, w3_ref, b3_ref, o_ref):
    dn = (((0,), (0,)), ((), ()))
    h = lax.dot_general(ue_ref[...], w1u_ref[...], dn,
                        preferred_element_type=jnp.float32)
    h += lax.dot_general(ie_ref[...], w1i_ref[...], dn,
                         preferred_element_type=jnp.float32)
    h = jnp.maximum(h + b1_ref[...], 0.0)
    h = jnp.dot(h, w2_ref[...], preferred_element_type=jnp.float32)
    h = jnp.maximum(h + b2_ref[...], 0.0)
    z = lax.dot_general(w3_ref[...], h, (((0,), (1,)), ((), ())),
                        preferred_element_type=jnp.float32)
    z = z + b3_ref[...]
    o_ref[...] = 1.0 / (1.0 + jnp.exp(-z))


def _tc_mlp(ue_t, ie_t, W1u, W1i, b1, W2, b2, W3, b3, block_b=8192):
    emb_dim, batch = ue_t.shape
    grid_n = batch // block_b
    full = lambda shape: pl.BlockSpec(shape, lambda i: (0,) * len(shape))
    return pl.pallas_call(
        _mlp_kernel_body,
        grid=(grid_n,),
        in_specs=[
            pl.BlockSpec((emb_dim, block_b), lambda i: (0, i)),
            pl.BlockSpec((emb_dim, block_b), lambda i: (0, i)),
            full(W1u.shape),
            full(W1i.shape),
            full(b1.shape),
            full(W2.shape),
            full(b2.shape),
            full(W3.shape),
            full(b3.shape),
        ],
        out_specs=pl.BlockSpec((1, block_b), lambda i: (0, i)),
        out_shape=jax.ShapeDtypeStruct((1, batch), jnp.float32),
    )(ue_t, ie_t, W1u, W1i, b1, W2, b2, W3, b3)


def kernel(user_input, item_input, user_table, item_table,
           W1, b1, W2, b2, W3, b3):
    emb_dim = user_table.shape[1]

    uidx = user_input.astype(jnp.int32)
    iidx = item_input.astype(jnp.int32)

    ue_t, ie_t = _sc_gather(uidx, iidx, user_table.T, item_table.T)

    W1u = W1[:emb_dim]
    W1i = W1[emb_dim:]
    out_t = _tc_mlp(ue_t, ie_t, W1u, W1i,
                    b1.reshape(1, -1), W2, b2.reshape(1, -1),
                    W3, b3.reshape(1, -1))
    return out_t.T
